# Initial kernel scaffold; baseline (speedup 1.0000x reference)
import jax
import jax.numpy as jnp
from jax.experimental import pallas as pl


def kernel(target, Wx, Wy):
    raise NotImplementedError("write your pallas kernel here")


# speedup vs baseline: 2.9550x; 2.9550x over previous
import functools

import jax
import jax.numpy as jnp
from jax import lax
from jax.experimental import pallas as pl
from jax.experimental.pallas import tpu as pltpu
from jax.experimental.pallas import tpu_sc as plsc

_H = 128
_D = 64
_V = 1024


def _pos_encoding(seq_len, d_model):
    pos = jnp.arange(seq_len, dtype=jnp.float32)[:, None]
    dim = jnp.arange(0, d_model, 2, dtype=jnp.float32)
    angle = pos / jnp.power(10000.0, dim / float(d_model))
    res = jnp.zeros((seq_len, d_model), dtype=jnp.float32)
    res = res.at[:, 0::2].set(jnp.sin(angle))
    res = res.at[:, 1::2].set(jnp.cos(angle))
    return res


def kernel(target, Wx, Wy):
    B, T, N, _ = target.shape
    K = B * N * T * 2

    ws = jnp.concatenate([Wx, Wy], axis=0)
    j = (jnp.transpose(target, (0, 2, 1, 3)).reshape(-1, 2)
         + jnp.array([0, _V], jnp.int32)).reshape(-1)
    pe2 = _pos_encoding(T, _H).reshape(2 * T, _D)

    info = plsc.get_sparse_core_info()
    nw = info.num_cores * info.num_subcores
    per_w = K // nw
    ch = 4 * 2 * T
    n_ch = per_w // ch

    mesh = plsc.VectorSubcoreMesh(core_axis_name="c", subcore_axis_name="s")

    @functools.partial(
        pl.kernel,
        out_type=jax.ShapeDtypeStruct((K, _D), jnp.float32),
        mesh=mesh,
        compiler_params=pltpu.CompilerParams(use_tc_tiling_on_sc=False),
        scratch_types=[
            pltpu.VMEM((ch,), jnp.int32),
            pltpu.VMEM((ch, _D), jnp.float32),
            pltpu.VMEM((ch, _D), jnp.float32),
            pltpu.SemaphoreType.DMA,
        ],
    )
    def emb_kernel(ws_hbm, j_hbm, pe_hbm, out_hbm, idx_v, rows_v, pe_v, sem):
        wid = lax.axis_index("s") * info.num_cores + lax.axis_index("c")
        base = wid * per_w
        for g in range(ch // (2 * T)):
            pltpu.sync_copy(pe_hbm, pe_v.at[pl.ds(g * 2 * T, 2 * T)])

        @pl.loop(0, n_ch)
        def _chunk(ci):
            off = base + ci * ch
            pltpu.sync_copy(j_hbm.at[pl.ds(off, ch)], idx_v)
            pltpu.async_copy(ws_hbm.at[idx_v], rows_v, sem).wait()

            @pl.loop(0, ch)
            def _row(r):
                for q in range(_D // 16):
                    sl = pl.ds(q * 16, 16)
                    rows_v[r, sl] = rows_v[r, sl] + pe_v[r, sl]

            pltpu.sync_copy(rows_v, out_hbm.at[pl.ds(off, ch)])

    out = emb_kernel(ws, j, pe2)
    return out.reshape(B * N, T, _H)
